# baseline (speedup 1.0000x reference)
import functools

import jax
import jax.numpy as jnp
from jax import lax
from jax.experimental import pallas as pl
from jax.experimental.pallas import tpu as pltpu
from jax.experimental.pallas import tpu_sc as plsc

N = 10000
D = 128
NC = 2
NS = 16
NW = NC * NS
CHUNK = 128
ACC_ROWS = 10112
RPT = ACC_ROWS // NS
BR = 1000
EPS = 1e-5


def _sc_agg(h, srcf, dst3, zf, K):
    mesh = plsc.VectorSubcoreMesh(core_axis_name="c", subcore_axis_name="s")

    @functools.partial(
        pl.kernel,
        out_type=jax.ShapeDtypeStruct((NC, ACC_ROWS, D), jnp.float32),
        mesh=mesh,
        scratch_types=(
            pltpu.VMEM((1, CHUNK), jnp.int32),
            pltpu.VMEM((1, CHUNK), jnp.int32),
            pltpu.VMEM((K, CHUNK), jnp.int32),
            pltpu.VMEM((CHUNK, D), jnp.float32),
            pltpu.VMEM((CHUNK, D), jnp.float32),
            pltpu.VMEM_SHARED((ACC_ROWS, D), jnp.float32),
            pltpu.SemaphoreType.DMA,
            pltpu.SemaphoreType.DMA,
            pltpu.SemaphoreType.DMA,
            pltpu.SemaphoreType.DMA,
            pltpu.SemaphoreType.DMA,
            pltpu.SemaphoreType.DMA,
        ),
    )
    def k(h_hbm, src_hbm, dst_hbm, zf_hbm, agg_hbm,
          src_a, src_b, dst_v, rows_a, rows_b, acc_s,
          sem_ia, sem_ib, sem_ga, sem_gb, sem_sa, sem_sb):
        c = lax.axis_index("c")
        s = lax.axis_index("s")
        w = c * NS + s
        base = s * RPT
        pltpu.sync_copy(dst_hbm.at[w], dst_v)
        pltpu.sync_copy(zf_hbm.at[pl.ds(base, RPT)],
                        acc_s.at[pl.ds(base, RPT)])
        plsc.subcore_barrier()

        pltpu.async_copy(src_hbm.at[w * K], src_a, sem_ia)
        pltpu.async_copy(src_hbm.at[w * K + 1], src_b, sem_ib)
        pltpu.make_async_copy(src_hbm.at[w * K], src_a, sem_ia).wait()
        pltpu.async_copy(h_hbm.at[src_a.at[0]], rows_a, sem_ga)

        def step(j, srcb, rowsb, othersrc, otherrows, sem_i, sem_io,
                 sem_g, sem_go, sem_s, sem_so):
            pltpu.make_async_copy(h_hbm.at[srcb.at[0]], rowsb, sem_g).wait()
            pltpu.async_copy(rowsb, acc_s.at[dst_v.at[j]], sem_s, add=True)

            @pl.when(j + 1 < K)
            def _():
                @pl.when(j > 0)
                def _():
                    pltpu.make_async_copy(otherrows,
                                          acc_s.at[dst_v.at[j]],
                                          sem_so).wait()
                pltpu.make_async_copy(src_hbm.at[w * K], othersrc,
                                      sem_io).wait()
                pltpu.async_copy(h_hbm.at[othersrc.at[0]], otherrows, sem_go)

            @pl.when(j + 2 < K)
            def _():
                pltpu.async_copy(src_hbm.at[w * K + j + 2], srcb, sem_i)

        def loop(jj, carry):
            j = jj * 2
            step(j, src_a, rows_a, src_b, rows_b,
                 sem_ia, sem_ib, sem_ga, sem_gb, sem_sa, sem_sb)
            step(j + 1, src_b, rows_b, src_a, rows_a,
                 sem_ib, sem_ia, sem_gb, sem_ga, sem_sb, sem_sa)
            return carry

        lax.fori_loop(0, K // 2, loop, 0)
        pltpu.make_async_copy(rows_a, acc_s.at[dst_v.at[0]], sem_sa).wait()
        pltpu.make_async_copy(rows_b, acc_s.at[dst_v.at[0]], sem_sb).wait()
        plsc.subcore_barrier()
        pltpu.sync_copy(acc_s.at[pl.ds(base, RPT)],
                        agg_hbm.at[c, pl.ds(base, RPT)])

    return k(h, srcf, dst3, zf)


def _sc_deg(dst3, ones, zf, K):
    mesh = plsc.VectorSubcoreMesh(core_axis_name="c", subcore_axis_name="s")

    @functools.partial(
        pl.kernel,
        out_type=jax.ShapeDtypeStruct((NC, ACC_ROWS, D), jnp.float32),
        mesh=mesh,
        scratch_types=(
            pltpu.VMEM((K, CHUNK), jnp.int32),
            pltpu.VMEM((CHUNK, D), jnp.float32),
            pltpu.VMEM_SHARED((ACC_ROWS, D), jnp.float32),
            pltpu.SemaphoreType.DMA,
        ),
    )
    def k(dst_hbm, ones_hbm, zf_hbm, deg_hbm, dst_v, ones_v, deg_s, sem):
        c = lax.axis_index("c")
        s = lax.axis_index("s")
        w = c * NS + s
        base = s * RPT
        pltpu.sync_copy(dst_hbm.at[w], dst_v)
        pltpu.sync_copy(ones_hbm, ones_v)
        pltpu.sync_copy(zf_hbm.at[pl.ds(base, RPT)],
                        deg_s.at[pl.ds(base, RPT)])
        plsc.subcore_barrier()

        def loop(j, carry):
            pltpu.async_copy(ones_v, deg_s.at[dst_v.at[j]], sem, add=True)

            @pl.when(j >= 8)
            def _():
                pltpu.make_async_copy(ones_v, deg_s.at[dst_v.at[0]],
                                      sem).wait()
            return carry

        lax.fori_loop(0, K, loop, 0)

        def drain(j, carry):
            pltpu.make_async_copy(ones_v, deg_s.at[dst_v.at[0]], sem).wait()
            return carry

        lax.fori_loop(0, 8, drain, 0)
        plsc.subcore_barrier()
        pltpu.sync_copy(deg_s.at[pl.ds(base, RPT)],
                        deg_hbm.at[c, pl.ds(base, RPT)])

    return k(dst3, ones, zf)


def _log1p_body(x_ref, o_ref):
    o_ref[...] = jnp.log(x_ref[...] + 1.0)


def _tc_log1p(x):
    return pl.pallas_call(
        _log1p_body,
        grid=(N // BR,),
        in_specs=[pl.BlockSpec((BR, D), lambda i: (i, 0))],
        out_specs=pl.BlockSpec((BR, D), lambda i: (i, 0)),
        out_shape=jax.ShapeDtypeStruct((N, D), jnp.float32),
    )(x)


def _stats_update(i, y, st_ref):
    @pl.when(i == 0)
    def _():
        st_ref[...] = jnp.zeros_like(st_ref)

    st_ref[...] += jnp.concatenate(
        [jnp.sum(y, axis=0, keepdims=True),
         jnp.broadcast_to(jnp.sum(y * y), (1, D))], axis=0)


def _mm1_body(h_ref, a_ref, d_ref, ws_ref, wn_ref, b_ref, y_ref, st_ref):
    i = pl.program_id(0)
    a = a_ref[0] + a_ref[1]
    deg = (d_ref[0] + d_ref[1])[:, 0:1]
    hn = a / jnp.maximum(deg, 1.0)
    z = jnp.dot(h_ref[...], ws_ref[...], preferred_element_type=jnp.float32)
    z += jnp.dot(hn, wn_ref[...], preferred_element_type=jnp.float32)
    y = jnp.maximum(z + b_ref[...], 0.0)
    y_ref[...] = y
    _stats_update(i, y, st_ref)


def _mm2_body(h_ref, a_ref, d_ref, ws_ref, wn_ref, b_ref, st_in_ref,
              y_ref, st_ref):
    i = pl.program_id(0)
    st = st_in_ref[...]
    mu = st[0:1] / N
    var = st[1, 0] / N - jnp.sum(mu * mu)
    cc = lax.rsqrt(var + EPS)
    a = a_ref[0] + a_ref[1]
    deg = (d_ref[0] + d_ref[1])[:, 0:1]
    h1 = (h_ref[...] - mu) * cc
    hn = (a - deg * mu) * cc / jnp.maximum(deg, 1.0)
    z = jnp.dot(h1, ws_ref[...], preferred_element_type=jnp.float32)
    z += jnp.dot(hn, wn_ref[...], preferred_element_type=jnp.float32)
    y = jnp.maximum(z + b_ref[...], 0.0)
    y_ref[...] = y
    _stats_update(i, y, st_ref)


def _full(shape):
    return pl.BlockSpec(shape, lambda i: tuple(0 for _ in shape))


def _tc_mm(body, h, aggp, degp, Ws, Wn, b, st_in=None):
    in_specs = [
        pl.BlockSpec((BR, D), lambda i: (i, 0)),
        pl.BlockSpec((NC, BR, D), lambda i: (0, i, 0)),
        pl.BlockSpec((NC, BR, D), lambda i: (0, i, 0)),
        _full((D, D)),
        _full((D, D)),
        _full((1, D)),
    ]
    args = [h, aggp, degp, Ws, Wn, b.reshape(1, D)]
    if st_in is not None:
        in_specs.append(_full((2, D)))
        args.append(st_in)
    return pl.pallas_call(
        body,
        grid=(N // BR,),
        in_specs=in_specs,
        out_specs=(pl.BlockSpec((BR, D), lambda i: (i, 0)),
                   _full((2, D))),
        out_shape=(jax.ShapeDtypeStruct((N, D), jnp.float32),
                   jax.ShapeDtypeStruct((2, D), jnp.float32)),
    )(*args)


def _norm_body(y_ref, st_ref, o_ref):
    st = st_ref[...]
    mu = st[0:1] / N
    var = st[1, 0] / N - jnp.sum(mu * mu)
    cc = lax.rsqrt(var + EPS)
    o_ref[...] = (y_ref[...] - mu) * cc


def _tc_norm(y, st):
    return pl.pallas_call(
        _norm_body,
        grid=(N // BR,),
        in_specs=[pl.BlockSpec((BR, D), lambda i: (i, 0)), _full((2, D))],
        out_specs=pl.BlockSpec((BR, D), lambda i: (i, 0)),
        out_shape=jax.ShapeDtypeStruct((N, D), jnp.float32),
    )(y, st)


def kernel(x, edge_index, W_self0, W_neigh0, b0, W_self1, W_neigh1, b1):
    E = edge_index.shape[1]
    K = -(-E // (NW * CHUNK))
    K += K % 2
    e_pad = NW * K * CHUNK
    pad = e_pad - E
    pid = jnp.arange(pad, dtype=jnp.int32)
    srcf = jnp.concatenate(
        [edge_index[0], pid % N]).reshape(NW * K, 1, CHUNK)
    dst3 = jnp.concatenate(
        [edge_index[1], N + pid % (ACC_ROWS - N)]).reshape(NW, K, CHUNK)
    zf = jnp.zeros((ACC_ROWS, D), jnp.float32)
    ones = jnp.ones((CHUNK, D), jnp.float32)

    h0 = _tc_log1p(x)
    degp = _sc_deg(dst3, ones, zf, K)
    agg0 = _sc_agg(h0, srcf, dst3, zf, K)
    y1, st1 = _tc_mm(_mm1_body, h0, agg0, degp, W_self0, W_neigh0, b0)
    agg1 = _sc_agg(y1, srcf, dst3, zf, K)
    y2, st2 = _tc_mm(_mm2_body, y1, agg1, degp, W_self1, W_neigh1, b1,
                     st_in=st1)
    return _tc_norm(y2, st2)


# speedup vs baseline: 1.0360x; 1.0360x over previous
import functools

import jax
import jax.numpy as jnp
from jax import lax
from jax.experimental import pallas as pl
from jax.experimental.pallas import tpu as pltpu
from jax.experimental.pallas import tpu_sc as plsc

N = 10000
D = 128
NC = 2
NS = 16
NW = NC * NS
CHUNK = 128
ACC_ROWS = 10112
RPT = ACC_ROWS // NS
BR = 1000
EPS = 1e-5


def _sc_agg(h, srcf, dst3, zf, K):
    mesh = plsc.VectorSubcoreMesh(core_axis_name="c", subcore_axis_name="s")

    @functools.partial(
        pl.kernel,
        out_type=jax.ShapeDtypeStruct((NC, ACC_ROWS, D), jnp.float32),
        mesh=mesh,
        scratch_types=(
            pltpu.VMEM((1, CHUNK), jnp.int32),
            pltpu.VMEM((1, CHUNK), jnp.int32),
            pltpu.VMEM((K, CHUNK), jnp.int32),
            pltpu.VMEM((CHUNK, D), jnp.float32),
            pltpu.VMEM((CHUNK, D), jnp.float32),
            pltpu.VMEM_SHARED((ACC_ROWS, D), jnp.float32),
            pltpu.SemaphoreType.DMA,
            pltpu.SemaphoreType.DMA,
            pltpu.SemaphoreType.DMA,
            pltpu.SemaphoreType.DMA,
        ),
    )
    def k(h_hbm, src_hbm, dst_hbm, zf_hbm, agg_hbm,
          src_a, src_b, dst_v, rows_a, rows_b, acc_s,
          sem_ia, sem_ib, sem_ga, sem_gb):
        c = lax.axis_index("c")
        s = lax.axis_index("s")
        w = c * NS + s
        base = s * RPT
        pltpu.sync_copy(dst_hbm.at[w], dst_v)
        pltpu.sync_copy(zf_hbm.at[pl.ds(base, RPT)],
                        acc_s.at[pl.ds(base, RPT)])
        plsc.subcore_barrier()

        pltpu.async_copy(src_hbm.at[w * K], src_a, sem_ia)
        pltpu.async_copy(src_hbm.at[w * K + 1], src_b, sem_ib)
        pltpu.make_async_copy(src_hbm.at[w * K], src_a, sem_ia).wait()
        pltpu.async_copy(h_hbm.at[src_a.at[0]], rows_a, sem_ga)

        def step(j, srcb, rowsb, othersrc, otherrows, sem_i, sem_io,
                 sem_g, sem_go):
            @pl.when(j + 1 < K)
            def _():
                pltpu.make_async_copy(src_hbm.at[w * K], othersrc,
                                      sem_io).wait()
                pltpu.async_copy(h_hbm.at[othersrc.at[0]], otherrows, sem_go)

            pltpu.make_async_copy(h_hbm.at[srcb.at[0]], rowsb, sem_g).wait()
            pltpu.sync_copy(rowsb, acc_s.at[dst_v.at[j]], add=True)

            @pl.when(j + 2 < K)
            def _():
                pltpu.async_copy(src_hbm.at[w * K + j + 2], srcb, sem_i)

        def loop(jj, carry):
            j = jj * 2
            step(j, src_a, rows_a, src_b, rows_b,
                 sem_ia, sem_ib, sem_ga, sem_gb)
            step(j + 1, src_b, rows_b, src_a, rows_a,
                 sem_ib, sem_ia, sem_gb, sem_ga)
            return carry

        lax.fori_loop(0, K // 2, loop, 0)
        plsc.subcore_barrier()
        pltpu.sync_copy(acc_s.at[pl.ds(base, RPT)],
                        agg_hbm.at[c, pl.ds(base, RPT)])

    return k(h, srcf, dst3, zf)


def _sc_deg(dst3, ones, zb, K):
    mesh = plsc.VectorSubcoreMesh(core_axis_name="c", subcore_axis_name="s")

    @functools.partial(
        pl.kernel,
        out_type=jax.ShapeDtypeStruct((NC, ACC_ROWS, D), jnp.float32),
        mesh=mesh,
        scratch_types=(
            pltpu.VMEM((K, CHUNK), jnp.int32),
            pltpu.VMEM((CHUNK, D), jnp.float32),
            pltpu.VMEM_SHARED((ACC_ROWS, D), jnp.float32),
            pltpu.SemaphoreType.DMA,
        ),
    )
    def k(dst_hbm, ones_hbm, zf_hbm, deg_hbm, dst_v, ones_v, deg_s, sem):
        c = lax.axis_index("c")
        s = lax.axis_index("s")
        w = c * NS + s
        base = s * RPT
        pltpu.sync_copy(dst_hbm.at[w], dst_v)
        pltpu.sync_copy(ones_hbm, ones_v)
        pltpu.sync_copy(zf_hbm.at[pl.ds(base, RPT)],
                        deg_s.at[pl.ds(base, RPT)])
        plsc.subcore_barrier()

        def loop(j, carry):
            pltpu.async_copy(ones_v, deg_s.at[dst_v.at[j]], sem, add=True)

            @pl.when(j >= 8)
            def _():
                pltpu.make_async_copy(ones_v, deg_s.at[dst_v.at[0]],
                                      sem).wait()
            return carry

        lax.fori_loop(0, K, loop, 0)

        def drain(j, carry):
            pltpu.make_async_copy(ones_v, deg_s.at[dst_v.at[0]], sem).wait()
            return carry

        lax.fori_loop(0, 8, drain, 0)
        plsc.subcore_barrier()
        pltpu.sync_copy(deg_s.at[pl.ds(base, RPT)],
                        deg_hbm.at[c, pl.ds(base, RPT)])

    return k(dst3, ones, zb)


def _log1p_body(x_ref, o_ref):
    o_ref[...] = jnp.log(x_ref[...] + 1.0)


def _tc_log1p(x):
    return pl.pallas_call(
        _log1p_body,
        grid=(N // BR,),
        in_specs=[pl.BlockSpec((BR, D), lambda i: (i, 0))],
        out_specs=pl.BlockSpec((BR, D), lambda i: (i, 0)),
        out_shape=jax.ShapeDtypeStruct((N, D), jnp.float32),
    )(x)


def _stats_update(i, y, st_ref):
    @pl.when(i == 0)
    def _():
        st_ref[...] = jnp.zeros_like(st_ref)

    st_ref[...] += jnp.concatenate(
        [jnp.sum(y, axis=0, keepdims=True),
         jnp.broadcast_to(jnp.sum(y * y), (1, D))], axis=0)


def _mm1_body(h_ref, a_ref, d_ref, ws_ref, wn_ref, b_ref, y_ref, st_ref):
    i = pl.program_id(0)
    a = a_ref[0] + a_ref[1]
    deg = d_ref[...]
    hn = a / jnp.maximum(deg, 1.0)
    z = jnp.dot(h_ref[...], ws_ref[...], preferred_element_type=jnp.float32)
    z += jnp.dot(hn, wn_ref[...], preferred_element_type=jnp.float32)
    y = jnp.maximum(z + b_ref[...], 0.0)
    y_ref[...] = y
    _stats_update(i, y, st_ref)


def _mm2_body(h_ref, a_ref, d_ref, ws_ref, wn_ref, b_ref, st_in_ref,
              y_ref, st_ref):
    i = pl.program_id(0)
    st = st_in_ref[...]
    mu = st[0:1] / N
    var = st[1, 0] / N - jnp.sum(mu * mu)
    cc = lax.rsqrt(var + EPS)
    a = a_ref[0] + a_ref[1]
    deg = d_ref[...]
    h1 = (h_ref[...] - mu) * cc
    hn = (a - deg * mu) * cc / jnp.maximum(deg, 1.0)
    z = jnp.dot(h1, ws_ref[...], preferred_element_type=jnp.float32)
    z += jnp.dot(hn, wn_ref[...], preferred_element_type=jnp.float32)
    y = jnp.maximum(z + b_ref[...], 0.0)
    y_ref[...] = y
    _stats_update(i, y, st_ref)


def _full(shape):
    return pl.BlockSpec(shape, lambda i: tuple(0 for _ in shape))


def _tc_mm(body, h, aggp, degp, Ws, Wn, b, st_in=None):
    in_specs = [
        pl.BlockSpec((BR, D), lambda i: (i, 0)),
        pl.BlockSpec((NC, BR, D), lambda i: (0, i, 0)),
        pl.BlockSpec((BR, 1), lambda i: (i, 0)),
        _full((D, D)),
        _full((D, D)),
        _full((1, D)),
    ]
    args = [h, aggp, degp, Ws, Wn, b.reshape(1, D)]
    if st_in is not None:
        in_specs.append(_full((2, D)))
        args.append(st_in)
    return pl.pallas_call(
        body,
        grid=(N // BR,),
        in_specs=in_specs,
        out_specs=(pl.BlockSpec((BR, D), lambda i: (i, 0)),
                   _full((2, D))),
        out_shape=(jax.ShapeDtypeStruct((N, D), jnp.float32),
                   jax.ShapeDtypeStruct((2, D), jnp.float32)),
    )(*args)


def _norm_body(y_ref, st_ref, o_ref):
    st = st_ref[...]
    mu = st[0:1] / N
    var = st[1, 0] / N - jnp.sum(mu * mu)
    cc = lax.rsqrt(var + EPS)
    o_ref[...] = (y_ref[...] - mu) * cc


def _tc_norm(y, st):
    return pl.pallas_call(
        _norm_body,
        grid=(N // BR,),
        in_specs=[pl.BlockSpec((BR, D), lambda i: (i, 0)), _full((2, D))],
        out_specs=pl.BlockSpec((BR, D), lambda i: (i, 0)),
        out_shape=jax.ShapeDtypeStruct((N, D), jnp.float32),
    )(y, st)


def kernel(x, edge_index, W_self0, W_neigh0, b0, W_self1, W_neigh1, b1):
    E = edge_index.shape[1]
    K = -(-E // (NW * CHUNK))
    K += K % 2
    e_pad = NW * K * CHUNK
    pad = e_pad - E
    pid = jnp.arange(pad, dtype=jnp.int32)
    srcf = jnp.concatenate(
        [edge_index[0], pid % N]).reshape(NW * K, 1, CHUNK)
    dst3 = jnp.concatenate(
        [edge_index[1], N + pid % (ACC_ROWS - N)]).reshape(NW, K, CHUNK)
    zf = jnp.zeros((ACC_ROWS, D), jnp.float32)
    zb = jnp.zeros((ACC_ROWS, D), jnp.float32)
    ones = jnp.ones((CHUNK, D), jnp.float32)

    h0 = _tc_log1p(x)
    degp = _sc_deg(dst3, ones, zb, K)
    deg = degp[0, :N, :1] + degp[1, :N, :1]
    agg0 = _sc_agg(h0, srcf, dst3, zf, K)
    y1, st1 = _tc_mm(_mm1_body, h0, agg0, deg, W_self0, W_neigh0, b0)
    agg1 = _sc_agg(y1, srcf, dst3, zf, K)
    y2, st2 = _tc_mm(_mm2_body, y1, agg1, deg, W_self1, W_neigh1, b1,
                     st_in=st1)
    return _tc_norm(y2, st2)
